# Initial kernel scaffold; baseline (speedup 1.0000x reference)
import jax
import jax.numpy as jnp
from jax.experimental import pallas as pl


def kernel(h, edge_index, W1, b1):
    raise NotImplementedError("write your pallas kernel here")


# speedup vs baseline: 29.9025x; 29.9025x over previous
import functools

import jax
import jax.numpy as jnp
from jax import lax
from jax.experimental import pallas as pl
from jax.experimental.pallas import tpu as pltpu
from jax.experimental.pallas import tpu_sc as plsc

N_NODES = 10000
N_EDGES = 320000
D_FEAT = 128
LANES = 16


def _matvec_body(h_ref, w_ref, b_ref, out_ref):
    out_ref[...] = (
        jnp.dot(h_ref[...], w_ref[...], preferred_element_type=jnp.float32)
        + b_ref[...]
    )


def _node_scalars(h, W1, b1):
    w2 = W1.reshape(2, D_FEAT).T
    bias = jnp.concatenate([jnp.zeros((1,), jnp.float32), b1]).reshape(1, 2)
    return pl.pallas_call(
        _matvec_body,
        out_shape=jax.ShapeDtypeStruct((N_NODES, 2), jnp.float32),
    )(h, w2, bias)


def _make_edge_kernel():
    info = plsc.get_sparse_core_info()
    nc, ns = info.num_cores, info.num_subcores
    nw = nc * ns
    per_w = N_EDGES // nw
    n_iter = per_w // LANES

    mesh = plsc.VectorSubcoreMesh(core_axis_name="c", subcore_axis_name="s")

    @functools.partial(
        pl.kernel,
        mesh=mesh,
        out_type=jax.ShapeDtypeStruct((N_EDGES,), jnp.float32),
        compiler_params=pltpu.CompilerParams(needs_layout_passes=False),
        scratch_types=[
            pltpu.VMEM((2 * N_NODES,), jnp.float32),
            pltpu.VMEM((per_w,), jnp.int32),
            pltpu.VMEM((per_w,), jnp.int32),
            pltpu.VMEM((per_w,), jnp.float32),
            pltpu.SemaphoreType.DMA,
        ],
    )
    def edge_kernel(t_hbm, src_hbm, dst_hbm, out_hbm, t_v, src_v, dst_v, out_v, sem):
        wid = lax.axis_index("s") * nc + lax.axis_index("c")
        base = wid * per_w
        c_t = pltpu.async_copy(t_hbm, t_v, sem)
        c_s = pltpu.async_copy(src_hbm.at[pl.ds(base, per_w)], src_v, sem)
        c_d = pltpu.async_copy(dst_hbm.at[pl.ds(base, per_w)], dst_v, sem)
        c_t.wait()
        c_s.wait()
        c_d.wait()

        def body(i, carry):
            off = i * LANES
            s_idx = src_v[pl.ds(off, LANES)]
            d_idx = dst_v[pl.ds(off, LANES)]
            pv = plsc.load_gather(t_v, [s_idx * 2])
            qv = plsc.load_gather(t_v, [d_idx * 2 + 1])
            out_v[pl.ds(off, LANES)] = pv + qv
            return carry

        lax.fori_loop(0, n_iter, body, 0)
        pltpu.sync_copy(out_v, out_hbm.at[pl.ds(base, per_w)])

    return edge_kernel


def kernel(h, edge_index, W1, b1):
    pq = _node_scalars(h, W1, b1)
    table = pq.reshape(2 * N_NODES)
    src = edge_index[0].astype(jnp.int32)
    dst = edge_index[1].astype(jnp.int32)
    score = _make_edge_kernel()(table, src, dst)
    return score.reshape(N_EDGES, 1)


# speedup vs baseline: 42.5612x; 1.4233x over previous
import functools

import jax
import jax.numpy as jnp
from jax import lax
from jax.experimental import pallas as pl
from jax.experimental.pallas import tpu as pltpu
from jax.experimental.pallas import tpu_sc as plsc

N_NODES = 10000
N_EDGES = 320000
D_FEAT = 128
LANES = 16


def _matvec_body(h_ref, w_ref, b_ref, out_ref):
    out_ref[...] = (
        lax.dot_general(
            w_ref[...],
            h_ref[...],
            (((1,), (1,)), ((), ())),
            preferred_element_type=jnp.float32,
        )
        + b_ref[...]
    )


def _node_scalars(h, W1, b1):
    w2 = W1.reshape(2, D_FEAT)
    bias = jnp.concatenate([jnp.zeros((1,), jnp.float32), b1]).reshape(2, 1)
    return pl.pallas_call(
        _matvec_body,
        out_shape=jax.ShapeDtypeStruct((2, N_NODES), jnp.float32),
    )(h, w2, bias)


def _make_edge_kernel():
    info = plsc.get_sparse_core_info()
    nc, ns = info.num_cores, info.num_subcores
    nw = nc * ns
    per_w = N_EDGES // nw

    mesh = plsc.VectorSubcoreMesh(core_axis_name="c", subcore_axis_name="s")

    @functools.partial(
        pl.kernel,
        mesh=mesh,
        out_type=jax.ShapeDtypeStruct((N_EDGES,), jnp.float32),
        compiler_params=pltpu.CompilerParams(needs_layout_passes=False),
        scratch_types=[
            pltpu.VMEM((N_NODES,), jnp.float32),
            pltpu.VMEM((N_NODES,), jnp.float32),
            pltpu.VMEM((per_w,), jnp.int32),
            pltpu.VMEM((per_w,), jnp.int32),
            pltpu.VMEM((per_w,), jnp.float32),
            pltpu.SemaphoreType.DMA,
        ],
    )
    def edge_kernel(pq_hbm, ei_hbm, out_hbm, p_v, q_v, src_v, dst_v, out_v, sem):
        wid = lax.axis_index("s") * nc + lax.axis_index("c")
        base = wid * per_w
        c_p = pltpu.async_copy(pq_hbm.at[pl.ds(0, N_NODES)], p_v, sem)
        c_q = pltpu.async_copy(pq_hbm.at[pl.ds(N_NODES, N_NODES)], q_v, sem)
        c_s = pltpu.async_copy(ei_hbm.at[pl.ds(base, per_w)], src_v, sem)
        c_d = pltpu.async_copy(ei_hbm.at[pl.ds(N_EDGES + base, per_w)], dst_v, sem)
        c_p.wait()
        c_q.wait()
        c_s.wait()
        c_d.wait()

        @plsc.parallel_loop(0, per_w, step=LANES, unroll=8)
        def body(off):
            s_idx = src_v[pl.ds(off, LANES)]
            d_idx = dst_v[pl.ds(off, LANES)]
            out_v[pl.ds(off, LANES)] = plsc.load_gather(
                p_v, [s_idx]
            ) + plsc.load_gather(q_v, [d_idx])

        pltpu.sync_copy(out_v, out_hbm.at[pl.ds(base, per_w)])

    return edge_kernel


def kernel(h, edge_index, W1, b1):
    pq = _node_scalars(h, W1, b1).reshape(2 * N_NODES)
    ei = edge_index
    if ei.dtype != jnp.int32:
        ei = ei.astype(jnp.int32)
    score = _make_edge_kernel()(pq, ei.reshape(2 * N_EDGES))
    return score.reshape(N_EDGES, 1)


# speedup vs baseline: 49.8161x; 1.1705x over previous
import functools

import jax
import jax.numpy as jnp
from jax import lax
from jax.experimental import pallas as pl
from jax.experimental.pallas import tpu as pltpu
from jax.experimental.pallas import tpu_sc as plsc

N_NODES = 10000
N_EDGES = 320000
D_FEAT = 128
LANES = 16


def _matvec_body(h_ref, w_ref, b_ref, out_ref):
    r = lax.dot_general(
        w_ref[...],
        h_ref[...],
        (((1,), (1,)), ((), ())),
        preferred_element_type=jnp.float32,
    )
    row = lax.broadcasted_iota(jnp.int32, (2, N_NODES), 0)
    out_ref[...] = r + jnp.where(row == 1, b_ref[...], jnp.float32(0.0))


def _node_scalars(h, W1, b1):
    w2 = W1.reshape(2, D_FEAT)
    return pl.pallas_call(
        _matvec_body,
        out_shape=jax.ShapeDtypeStruct((2, N_NODES), jnp.float32),
    )(h, w2, b1.reshape(1, 1))


def _make_edge_kernel():
    info = plsc.get_sparse_core_info()
    nc, ns = info.num_cores, info.num_subcores
    nw = nc * ns
    per_w = N_EDGES // nw
    chunk = per_w + 112

    mesh = plsc.VectorSubcoreMesh(core_axis_name="c", subcore_axis_name="s")

    @functools.partial(
        pl.kernel,
        mesh=mesh,
        out_type=jax.ShapeDtypeStruct((N_EDGES,), jnp.float32),
        compiler_params=pltpu.CompilerParams(needs_layout_passes=False),
        scratch_types=[
            pltpu.VMEM((2, N_NODES), jnp.float32),
            pltpu.VMEM((2, chunk), jnp.int32),
            pltpu.VMEM((per_w,), jnp.float32),
            pltpu.SemaphoreType.DMA,
        ],
    )
    def edge_kernel(pq_hbm, ei_hbm, out_hbm, pq_v, ei_v, out_v, sem):
        wid = lax.axis_index("s") * nc + lax.axis_index("c")
        base = wid * per_w
        aligned = pl.multiple_of(base - lax.rem(base, 128), 128)
        delta = base - aligned
        c_t = pltpu.async_copy(pq_hbm, pq_v, sem)
        c_e = pltpu.async_copy(ei_hbm.at[:, pl.ds(aligned, chunk)], ei_v, sem)
        c_t.wait()
        c_e.wait()

        zero16 = jnp.zeros((LANES,), jnp.int32)
        one16 = jnp.ones((LANES,), jnp.int32)

        @plsc.parallel_loop(0, per_w, step=LANES, unroll=8)
        def body(off):
            s_idx = ei_v[0, pl.ds(delta + off, LANES)]
            d_idx = ei_v[1, pl.ds(delta + off, LANES)]
            out_v[pl.ds(off, LANES)] = plsc.load_gather(
                pq_v, [zero16, s_idx]
            ) + plsc.load_gather(pq_v, [one16, d_idx])

        pltpu.sync_copy(out_v, out_hbm.at[pl.ds(base, per_w)])

    return edge_kernel


def kernel(h, edge_index, W1, b1):
    pq = _node_scalars(h, W1, b1)
    ei = edge_index
    if ei.dtype != jnp.int32:
        ei = ei.astype(jnp.int32)
    return _make_edge_kernel()(pq, ei).reshape(N_EDGES, 1)


# speedup vs baseline: 62.8374x; 1.2614x over previous
import functools

import jax
import jax.numpy as jnp
from jax import lax
from jax.experimental import pallas as pl
from jax.experimental.pallas import tpu as pltpu
from jax.experimental.pallas import tpu_sc as plsc

N_NODES = 10000
N_EDGES = 320000
D_FEAT = 128
LANES = 16
BLK = 128
N_BLOCKS = N_EDGES // BLK


def _matvec_body(h_ref, w_ref, b_ref, out_ref):
    r = lax.dot_general(
        w_ref[...],
        h_ref[...],
        (((1,), (1,)), ((), ())),
        preferred_element_type=jnp.float32,
    )
    row = lax.broadcasted_iota(jnp.int32, (2, N_NODES), 0)
    out_ref[...] = r + jnp.where(row == 1, b_ref[...], jnp.float32(0.0))


def _node_scalars(h, W1, b1):
    w2 = W1.reshape(2, D_FEAT)
    return pl.pallas_call(
        _matvec_body,
        out_shape=jax.ShapeDtypeStruct((2, N_NODES), jnp.float32),
    )(h, w2, b1.reshape(1, 1))


def _make_edge_kernel():
    info = plsc.get_sparse_core_info()
    nc, ns = info.num_cores, info.num_subcores
    nw = nc * ns
    base_blocks = N_BLOCKS // nw
    n_extra = N_BLOCKS - base_blocks * nw
    main_e = base_blocks * BLK
    max_e = main_e + BLK

    mesh = plsc.VectorSubcoreMesh(core_axis_name="c", subcore_axis_name="s")

    @functools.partial(
        pl.kernel,
        mesh=mesh,
        out_type=jax.ShapeDtypeStruct((1, N_EDGES), jnp.float32),
        compiler_params=pltpu.CompilerParams(needs_layout_passes=False),
        scratch_types=[
            pltpu.VMEM((2, N_NODES), jnp.float32),
            pltpu.VMEM((2, max_e), jnp.int32),
            pltpu.VMEM((max_e,), jnp.float32),
            pltpu.SemaphoreType.DMA,
        ],
    )
    def edge_kernel(pq_hbm, ei_hbm, out_hbm, pq_v, ei_v, out_v, sem):
        wid = lax.axis_index("s") * nc + lax.axis_index("c")
        extra = wid < n_extra
        start = pl.multiple_of(
            (base_blocks * wid + jnp.minimum(wid, n_extra)) * BLK, BLK
        )
        tail = pl.multiple_of(start + main_e, BLK)
        n_edges = jnp.where(extra, max_e, main_e)

        c_t = pltpu.async_copy(pq_hbm, pq_v, sem)
        c_e = pltpu.async_copy(
            ei_hbm.at[:, pl.ds(start, main_e)], ei_v.at[:, pl.ds(0, main_e)], sem
        )
        c_t.wait()
        c_e.wait()

        @pl.when(extra)
        def _():
            pltpu.sync_copy(
                ei_hbm.at[:, pl.ds(tail, BLK)], ei_v.at[:, pl.ds(main_e, BLK)]
            )

        zero16 = jnp.zeros((LANES,), jnp.int32)
        one16 = jnp.ones((LANES,), jnp.int32)

        @plsc.parallel_loop(0, n_edges, step=LANES, unroll=8)
        def body(off):
            s_idx = ei_v[0, pl.ds(off, LANES)]
            d_idx = ei_v[1, pl.ds(off, LANES)]
            out_v[pl.ds(off, LANES)] = plsc.load_gather(
                pq_v, [zero16, s_idx]
            ) + plsc.load_gather(pq_v, [one16, d_idx])

        pltpu.sync_copy(
            out_v.at[pl.ds(0, main_e)], out_hbm.at[0, pl.ds(start, main_e)]
        )

        @pl.when(extra)
        def _():
            pltpu.sync_copy(
                out_v.at[pl.ds(main_e, BLK)], out_hbm.at[0, pl.ds(tail, BLK)]
            )

    return edge_kernel


def kernel(h, edge_index, W1, b1):
    pq = _node_scalars(h, W1, b1)
    ei = edge_index
    if ei.dtype != jnp.int32:
        ei = ei.astype(jnp.int32)
    return _make_edge_kernel()(pq, ei).reshape(N_EDGES, 1)
